# baseline (speedup 1.0000x reference)
import functools

import jax
import jax.numpy as jnp
from jax import lax
from jax.experimental import pallas as pl
from jax.experimental.pallas import tpu as pltpu
from jax.experimental.pallas import tpu_sc as plsc

_BATCH = 4
_SEQ = 2048
_DIMS = 1024

_info = plsc.get_sparse_core_info()
_NC, _NS = _info.num_cores, _info.num_subcores
_NW = _NC * _NS
_ROWS_PER_W = _SEQ // _NW
_CHUNKS = (32, 24, 8)
_R_SH = 56


def _make_gather_broadcast():
  mesh = plsc.VectorSubcoreMesh(core_axis_name="c", subcore_axis_name="s")

  @functools.partial(
      pl.kernel,
      mesh=mesh,
      out_type=jax.ShapeDtypeStruct((_BATCH, _SEQ, _DIMS), jnp.float32),
      scratch_types=[
          pltpu.VMEM((_ROWS_PER_W,), jnp.int32),
          pltpu.VMEM((_ROWS_PER_W, _DIMS), jnp.float32),
          pltpu.VMEM_SHARED((_NS, _R_SH, _DIMS), jnp.float32),
          pltpu.SemaphoreType.DMA,
          pltpu.SemaphoreType.DMA,
          pltpu.SemaphoreType.DMA,
          pltpu.SemaphoreType.DMA,
      ],
  )
  def gather_broadcast(table_hbm, idx_hbm, out_hbm, idx_v, rows_v, shared,
                       sem_g, sem_d, sem_w, sem_s):
    cid = lax.axis_index("c")
    sid = lax.axis_index("s")
    wid = sid * _NC + cid
    base = wid * _ROWS_PER_W
    pltpu.sync_copy(idx_hbm.at[pl.ds(base, _ROWS_PER_W)], idx_v)

    offs = [sum(_CHUNKS[:i]) for i in range(len(_CHUNKS))]
    gathers = [
        pltpu.async_copy(table_hbm.at[idx_v.at[pl.ds(o, n)]],
                         rows_v.at[pl.ds(o, n)], sem_g)
        for o, n in zip(offs, _CHUNKS)
    ]

    writes = []
    deposits = []
    for i, (o, n) in enumerate(zip(offs, _CHUNKS)):
      gathers[i].wait()
      if o < _R_SH:
        deposits.append(
            pltpu.async_copy(rows_v.at[pl.ds(o, n)],
                             shared.at[sid, pl.ds(o, n)], sem_d))
        tgt_batches = range(2)
      else:
        tgt_batches = range(_BATCH)
      writes += [
          pltpu.async_copy(rows_v.at[pl.ds(o, n)],
                           out_hbm.at[b, pl.ds(base + o, n)], sem_w)
          for b in tgt_batches
      ]

    for i, (o, n) in enumerate(zip(offs[:len(deposits)],
                                   _CHUNKS[:len(deposits)])):
      deposits[i].wait()
      plsc.subcore_barrier()
      for drain_sid, b in ((0, 2), (1, 3)):
        @pl.when(sid == drain_sid)
        def _(o=o, n=n, b=b):
          chunk_drains = [
              pltpu.async_copy(
                  shared.at[s, pl.ds(o, n)],
                  out_hbm.at[b, pl.ds((s * _NC + cid) * _ROWS_PER_W + o, n)],
                  sem_s,
              )
              for s in range(_NS)
          ]
          for d in chunk_drains:
            d.wait()

    for w in writes:
      w.wait()

  return gather_broadcast


_gather_broadcast = _make_gather_broadcast()


def kernel(x, E_absolute_position, relative_index):
  del x
  return _gather_broadcast(E_absolute_position, relative_index)
